# baseline (speedup 1.0000x reference)
import functools

import jax
import jax.numpy as jnp
from jax import lax
from jax.experimental import pallas as pl
from jax.experimental.pallas import tpu as pltpu
from jax.experimental.pallas import tpu_sc as plsc

_NC = 2
_NS = 16
_NW = _NC * _NS
_L = 16

_C = 80


def _tc_table_body(z_ref, w_ref, b_ref, o1_ref, o2_ref):
    zw = lax.dot_general(
        z_ref[...], w_ref[...],
        dimension_numbers=(((1,), (1,)), ((), ())),
        preferred_element_type=jnp.float32,
    )
    o1_ref[...] = z_ref[...].astype(jnp.bfloat16)
    o2_ref[...] = ((1e-6 - b_ref[...]) - zw).astype(jnp.bfloat16)


def _make_tables(z, W, b):
    n, d = z.shape
    return pl.pallas_call(
        _tc_table_body,
        out_shape=[
            jax.ShapeDtypeStruct((n, d), jnp.bfloat16),
            jax.ShapeDtypeStruct((n, d), jnp.bfloat16),
        ],
    )(z, W, b.reshape(1, d))


def _sc_body(e_per_w, e0_hbm, e1_hbm, z_hbm, nzw_hbm, out_hbm,
             idx0_v, idx1_v, r0_v, out_v, sem_a, sem_b):
    wid = lax.axis_index("s") * _NC + lax.axis_index("c")
    base = wid * e_per_w

    pltpu.sync_copy(e0_hbm.at[pl.ds(base, e_per_w)], idx0_v)
    pltpu.sync_copy(e1_hbm.at[pl.ds(base, e_per_w)], idx1_v)

    n_groups = e_per_w // _C

    def issue_a(g):
        slot = lax.rem(g, 10)
        pltpu.async_copy(nzw_hbm.at[idx1_v.at[pl.ds(g * _C, _C)]],
                         r0_v.at[slot], sem_a.at[slot])

    def wait_a(g):
        slot = lax.rem(g, 10)
        pltpu.make_async_copy(nzw_hbm.at[idx1_v.at[pl.ds(0, _C)]],
                              r0_v.at[slot], sem_a.at[slot]).wait()

    def issue_b(g):
        slot = lax.rem(g, 10)
        pltpu.async_copy(z_hbm.at[idx0_v.at[pl.ds(g * _C, _C)]],
                         r0_v.at[slot], sem_b.at[slot], add=True)

    def wait_b(g):
        slot = lax.rem(g, 10)
        pltpu.make_async_copy(z_hbm.at[idx0_v.at[pl.ds(0, _C)]],
                              r0_v.at[slot], sem_b.at[slot]).wait()

    for _g in range(9):
        issue_a(_g)
    for _g in range(4):
        wait_a(_g)
        issue_b(_g)

    def group(g, carry):
        slot = lax.rem(g, 10)

        @pl.when(g + 9 < n_groups)
        def _():
            issue_a(g + 9)

        @pl.when(g + 4 < n_groups)
        def _():
            wait_a(g + 4)
            issue_b(g + 4)

        wait_b(g)
        off = g * _C
        lane = lax.iota(jnp.int32, _L)
        last_idx = jnp.full((_L,), _L - 1, dtype=jnp.int32)
        for s in range(_C // _L):
            vecsum = jnp.zeros((_L,), jnp.float32)
            for e in range(_L):
                ee = s * _L + e
                acc = None
                for k in range(128 // (2 * _L)):
                    d = r0_v[slot, ee, pl.ds(k * 2 * _L, 2 * _L)]
                    p = d * d
                    lo, hi = plsc.unpack(
                        p, format=plsc.PackFormat.INTERLEAVED)
                    acc = (lo + hi) if acc is None else (acc + lo + hi)
                cum = plsc.cumsum(acc)
                tot = jnp.take_along_axis(
                    cum, last_idx, axis=0, mode="promise_in_bounds")
                vecsum = jnp.where(lane == e, tot, vecsum)
            v = jnp.maximum(vecsum, 1e-30)
            i = lax.bitcast_convert_type(v, jnp.int32)
            i = 0x5F3759DF - lax.shift_right_arithmetic(i, 1)
            r = lax.bitcast_convert_type(i, jnp.float32)
            for _ in range(3):
                r = r * (1.5 - 0.5 * v * r * r)
            out_v[pl.ds(off + s * _L, _L)] = jnp.exp(-(v * r))
        return carry

    lax.fori_loop(0, n_groups, group, 0)

    pltpu.sync_copy(out_v, out_hbm.at[pl.ds(base, e_per_w)])


def _sc_distance(e0, e1, z_bf, nzw_bf):
    n_edges = e0.shape[0]
    assert n_edges % (_NW * _C) == 0
    e_per_w = n_edges // _NW
    mesh = plsc.VectorSubcoreMesh(core_axis_name="c", subcore_axis_name="s")
    k = pl.kernel(
        functools.partial(_sc_body, e_per_w),
        out_type=jax.ShapeDtypeStruct((n_edges,), jnp.float32),
        mesh=mesh,
        compiler_params=pltpu.CompilerParams(
            needs_layout_passes=False,
            use_tc_tiling_on_sc=False,
        ),
        scratch_types=[
            pltpu.VMEM((e_per_w,), jnp.int32),
            pltpu.VMEM((e_per_w,), jnp.int32),
            pltpu.VMEM((10, _C, 128), jnp.bfloat16),
            pltpu.VMEM((e_per_w,), jnp.float32),
            pltpu.SemaphoreType.DMA((10,)),
            pltpu.SemaphoreType.DMA((10,)),
        ],
    )
    return k(e0, e1, z_bf, nzw_bf)


def kernel(z, edge_index, W, b):
    e = edge_index.astype(jnp.int32)
    z_bf, nzw_bf = _make_tables(z, W, b)
    return _sc_distance(e[0], e[1], z_bf, nzw_bf)
